# baseline (speedup 1.0000x reference)
import functools

import jax
import jax.numpy as jnp
from jax import lax
from jax.experimental import pallas as pl
from jax.experimental.pallas import tpu as pltpu
from jax.experimental.pallas import tpu_sc as plsc

_NC, _NS, _L = 2, 16, 16
_NW = _NC * _NS
_B = 128


def _zero_vmem(ref, n):
    zeros = jnp.zeros((_L,), jnp.float32)

    @plsc.parallel_loop(0, n, step=_L, unroll=8)
    def _(i):
        ref[pl.ds(i, _L)] = zeros


def _rsqrt_newton(d):
    half = 0.5 * d
    yi = jnp.full((_L,), 0x5F3759DF, jnp.int32) - lax.shift_right_logical(
        plsc.bitcast(d, jnp.int32), jnp.full((_L,), 1, jnp.int32))
    y = plsc.bitcast(yi, jnp.float32)
    for _ in range(3):
        y = y * (1.5 - half * y * y)
    return y


@functools.cache
def _build(n, e):
    n_pad = ((n + (_L * _NS) - 1) // (_L * _NS)) * (_L * _NS)
    cols = n_pad // _NS
    nblk = e // _B
    bpt1 = nblk // _NS
    rem1 = nblk - bpt1 * _NS
    w1 = bpt1 * _B
    half = w1 // 2
    mesh = plsc.VectorSubcoreMesh(core_axis_name="c", subcore_axis_name="s",
                                  num_cores=_NC, num_subcores=_NS)
    sc_params = pltpu.CompilerParams(needs_layout_passes=False)
    nch = 6
    csz = w1 // nch

    @functools.partial(
        pl.kernel, mesh=mesh,
        out_type=(jax.ShapeDtypeStruct((_NW, n_pad), jnp.float32),
                  jax.ShapeDtypeStruct((n_pad,), jnp.float32),
                  jax.ShapeDtypeStruct((n_pad,), jnp.float32)),
        scratch_types=[pltpu.VMEM((2, w1 + _B), jnp.int32),
                       pltpu.VMEM((n_pad,), jnp.float32),
                       pltpu.VMEM((n_pad,), jnp.float32),
                       pltpu.VMEM((_NS, cols), jnp.float32),
                       pltpu.VMEM((cols,), jnp.float32),
                       pltpu.VMEM((cols,), jnp.float32),
                       pltpu.VMEM_SHARED((_NS, n_pad), jnp.float32),
                       pltpu.VMEM_SHARED((n_pad,), jnp.float32),
                       pltpu.SemaphoreType.DMA,
                       pltpu.SemaphoreType.DMA,
                       pltpu.SemaphoreType.DMA,
                       pltpu.SemaphoreType.DMA],
        compiler_params=sc_params,
    )
    def sc_call(ei_hbm, tpart_hbm, dis_hbm, inv_hbm,
                eb_v, hist_v, dis_v, blk_v, diss_v, invs_v,
                hist_sh, dis_sh, sem_a, sem_b, sem_c, sem_d):
        cid = lax.axis_index("c")
        sid = lax.axis_index("s")
        wid = sid * _NC + cid
        ones = jnp.ones((_L,), jnp.float32)
        sems = [sem_a, sem_b, sem_c, sem_d]

        def chunk_copy(k, sem):
            return pltpu.async_copy(
                ei_hbm.at[:, pl.ds(sid * w1 + k * csz, csz)],
                eb_v.at[:, pl.ds(k * csz, csz)], sem)

        cps = [chunk_copy(0, sems[0])]
        _zero_vmem(hist_v, n_pad)

        @pl.when(sid < rem1)
        def _():
            pltpu.sync_copy(ei_hbm.at[:, pl.ds((_NS * bpt1 + sid) * _B, _B)],
                            eb_v.at[:, pl.ds(w1, _B)])

        for k in range(nch):
            if k + 1 < nch:
                cps.append(chunk_copy(k + 1, sems[(k + 1) % len(sems)]))
            cps[k].wait()

            @plsc.parallel_loop(k * csz, (k + 1) * csz, step=_L, unroll=8)
            def _(i):
                plsc.addupdate_scatter(hist_v, [eb_v[1, pl.ds(i, _L)]], ones)

        @pl.when(sid < rem1)
        def _():
            @plsc.parallel_loop(0, _B, step=_L)
            def _(i):
                plsc.addupdate_scatter(hist_v, [eb_v[1, pl.ds(w1 + i, _L)]], ones)

        pltpu.sync_copy(hist_v, hist_sh.at[sid])
        plsc.subcore_barrier()

        pltpu.sync_copy(hist_sh.at[:, pl.ds(sid * cols, cols)], blk_v)

        @plsc.parallel_loop(0, cols, step=_L, unroll=2)
        def _(j):
            acc = blk_v[0, pl.ds(j, _L)]
            for r in range(1, _NS):
                acc = acc + blk_v[r, pl.ds(j, _L)]
            d = acc + 1.0
            y = _rsqrt_newton(d)
            diss_v[pl.ds(j, _L)] = y
            invs_v[pl.ds(j, _L)] = 1.0 / d

        pltpu.sync_copy(diss_v, dis_sh.at[pl.ds(sid * cols, cols)])

        @pl.when(cid == 0)
        def _():
            pltpu.sync_copy(diss_v, dis_hbm.at[pl.ds(sid * cols, cols)])
            pltpu.sync_copy(invs_v, inv_hbm.at[pl.ds(sid * cols, cols)])

        plsc.subcore_barrier()

        dis_cp = pltpu.async_copy(dis_sh, dis_v, sem_a)
        _zero_vmem(hist_v, n_pad)
        dis_cp.wait()
        base3 = cid * half

        @plsc.parallel_loop(0, half, step=_L, unroll=8)
        def _(i):
            w = plsc.load_gather(dis_v, [eb_v[1, pl.ds(base3 + i, _L)]])
            plsc.addupdate_scatter(hist_v, [eb_v[0, pl.ds(base3 + i, _L)]], w)

        @pl.when((sid < rem1) & (cid == sid % _NC))
        def _():
            @plsc.parallel_loop(0, _B, step=_L)
            def _(i):
                w = plsc.load_gather(dis_v, [eb_v[1, pl.ds(w1 + i, _L)]])
                plsc.addupdate_scatter(hist_v, [eb_v[0, pl.ds(w1 + i, _L)]], w)

        pltpu.sync_copy(hist_v, tpart_hbm.at[wid])

    def final_body(tpart_ref, dis_ref, inv_ref, x_ref, w_ref, b_ref, out_ref):
        t = jnp.sum(tpart_ref[...], axis=0)
        u = (dis_ref[...] * t + inv_ref[...])[:n].reshape(1, n)
        v = jnp.dot(u, x_ref[...], preferred_element_type=jnp.float32)
        out_ref[...] = (
            jnp.dot(v * (1.0 / n), w_ref[...], preferred_element_type=jnp.float32)
            + b_ref[...]
        )

    def final_call(tpart, dis, inv, x, w, b2):
        return pl.pallas_call(
            final_body,
            out_shape=jax.ShapeDtypeStruct((1, w.shape[1]), jnp.float32),
        )(tpart, dis, inv, x, w, b2)

    return sc_call, final_call


def kernel(x, edge_index, W, b):
    n = x.shape[0]
    e = edge_index.shape[1]
    sc_call, final_call = _build(n, e)
    tpart, dis, inv = sc_call(edge_index.astype(jnp.int32))
    return final_call(tpart, dis, inv, x, W, b.reshape(1, -1))


# speedup vs baseline: 1.0195x; 1.0195x over previous
import functools

import jax
import jax.numpy as jnp
from jax import lax
from jax.experimental import pallas as pl
from jax.experimental.pallas import tpu as pltpu
from jax.experimental.pallas import tpu_sc as plsc

_NC, _NS, _L = 2, 16, 16
_NW = _NC * _NS
_B = 128


def _zero_vmem(ref, n):
    zeros = jnp.zeros((_L,), jnp.float32)

    @plsc.parallel_loop(0, n, step=_L, unroll=8)
    def _(i):
        ref[pl.ds(i, _L)] = zeros


def _rsqrt_newton(d):
    half = 0.5 * d
    yi = jnp.full((_L,), 0x5F3759DF, jnp.int32) - lax.shift_right_logical(
        plsc.bitcast(d, jnp.int32), jnp.full((_L,), 1, jnp.int32))
    y = plsc.bitcast(yi, jnp.float32)
    for _ in range(3):
        y = y * (1.5 - half * y * y)
    return y


@functools.cache
def _build(n, e):
    n_pad = ((n + (_L * _NS) - 1) // (_L * _NS)) * (_L * _NS)
    cols = n_pad // _NS
    nblk = e // _B
    bpt1 = nblk // _NS
    rem1 = nblk - bpt1 * _NS
    w1 = bpt1 * _B
    half = w1 // 2
    mesh = plsc.VectorSubcoreMesh(core_axis_name="c", subcore_axis_name="s",
                                  num_cores=_NC, num_subcores=_NS)
    sc_params = pltpu.CompilerParams(needs_layout_passes=False)
    nch = 4
    csz = w1 // nch

    @functools.partial(
        pl.kernel, mesh=mesh,
        out_type=(jax.ShapeDtypeStruct((_NW, n_pad), jnp.float32),
                  jax.ShapeDtypeStruct((n_pad,), jnp.float32),
                  jax.ShapeDtypeStruct((n_pad,), jnp.float32)),
        scratch_types=[pltpu.VMEM((2, w1 + _B), jnp.int32),
                       pltpu.VMEM((n_pad,), jnp.float32),
                       pltpu.VMEM((n_pad,), jnp.float32),
                       pltpu.VMEM((_NS, cols), jnp.float32),
                       pltpu.VMEM((cols,), jnp.float32),
                       pltpu.VMEM((cols,), jnp.float32),
                       pltpu.VMEM_SHARED((_NS, n_pad), jnp.float32),
                       pltpu.VMEM_SHARED((n_pad,), jnp.float32),
                       pltpu.SemaphoreType.DMA,
                       pltpu.SemaphoreType.DMA,
                       pltpu.SemaphoreType.DMA,
                       pltpu.SemaphoreType.DMA],
        compiler_params=sc_params,
    )
    def sc_call(ei_hbm, tpart_hbm, dis_hbm, inv_hbm,
                eb_v, hist_v, dis_v, blk_v, diss_v, invs_v,
                hist_sh, dis_sh, sem_a, sem_b, sem_c, sem_d):
        cid = lax.axis_index("c")
        sid = lax.axis_index("s")
        wid = sid * _NC + cid
        ones = jnp.ones((_L,), jnp.float32)
        sems = [sem_a, sem_b, sem_c, sem_d]

        def chunk_copy(k, sem):
            return pltpu.async_copy(
                ei_hbm.at[:, pl.ds(sid * w1 + k * csz, csz)],
                eb_v.at[:, pl.ds(k * csz, csz)], sem)

        cps = [chunk_copy(0, sems[0])]
        _zero_vmem(hist_v, n_pad)

        @pl.when(sid < rem1)
        def _():
            pltpu.sync_copy(ei_hbm.at[:, pl.ds((_NS * bpt1 + sid) * _B, _B)],
                            eb_v.at[:, pl.ds(w1, _B)])

        for k in range(nch):
            if k + 1 < nch:
                cps.append(chunk_copy(k + 1, sems[(k + 1) % len(sems)]))
            cps[k].wait()

            @plsc.parallel_loop(k * csz, (k + 1) * csz, step=_L, unroll=8)
            def _(i):
                plsc.addupdate_scatter(hist_v, [eb_v[1, pl.ds(i, _L)]], ones)

        @pl.when(sid < rem1)
        def _():
            @plsc.parallel_loop(0, _B, step=_L)
            def _(i):
                plsc.addupdate_scatter(hist_v, [eb_v[1, pl.ds(w1 + i, _L)]], ones)

        pltpu.sync_copy(hist_v, hist_sh.at[sid])
        plsc.subcore_barrier()

        pltpu.sync_copy(hist_sh.at[:, pl.ds(sid * cols, cols)], blk_v)

        @plsc.parallel_loop(0, cols, step=_L, unroll=2)
        def _(j):
            acc = blk_v[0, pl.ds(j, _L)]
            for r in range(1, _NS):
                acc = acc + blk_v[r, pl.ds(j, _L)]
            d = acc + 1.0
            y = _rsqrt_newton(d)
            diss_v[pl.ds(j, _L)] = y
            invs_v[pl.ds(j, _L)] = 1.0 / d

        pltpu.sync_copy(diss_v, dis_sh.at[pl.ds(sid * cols, cols)])

        @pl.when(cid == 0)
        def _():
            pltpu.sync_copy(diss_v, dis_hbm.at[pl.ds(sid * cols, cols)])
            pltpu.sync_copy(invs_v, inv_hbm.at[pl.ds(sid * cols, cols)])

        plsc.subcore_barrier()

        dis_cp = pltpu.async_copy(dis_sh, dis_v, sem_a)
        _zero_vmem(hist_v, n_pad)
        dis_cp.wait()
        base3 = cid * half

        @plsc.parallel_loop(0, half, step=_L, unroll=8)
        def _(i):
            w = plsc.load_gather(dis_v, [eb_v[1, pl.ds(base3 + i, _L)]])
            plsc.addupdate_scatter(hist_v, [eb_v[0, pl.ds(base3 + i, _L)]], w)

        @pl.when((sid < rem1) & (cid == sid % _NC))
        def _():
            @plsc.parallel_loop(0, _B, step=_L)
            def _(i):
                w = plsc.load_gather(dis_v, [eb_v[1, pl.ds(w1 + i, _L)]])
                plsc.addupdate_scatter(hist_v, [eb_v[0, pl.ds(w1 + i, _L)]], w)

        pltpu.sync_copy(hist_v, tpart_hbm.at[wid])

    def final_body(tpart_ref, dis_ref, inv_ref, x_ref, w_ref, b_ref, out_ref):
        t = jnp.sum(tpart_ref[...], axis=0)
        u = (dis_ref[...] * t + inv_ref[...])[:n].reshape(1, n)
        v = jnp.dot(u, x_ref[...], preferred_element_type=jnp.float32)
        out_ref[...] = (
            jnp.dot(v * (1.0 / n), w_ref[...], preferred_element_type=jnp.float32)
            + b_ref[...]
        )

    def final_call(tpart, dis, inv, x, w, b2):
        return pl.pallas_call(
            final_body,
            out_shape=jax.ShapeDtypeStruct((1, w.shape[1]), jnp.float32),
        )(tpart, dis, inv, x, w, b2)

    return sc_call, final_call


def kernel(x, edge_index, W, b):
    n = x.shape[0]
    e = edge_index.shape[1]
    sc_call, final_call = _build(n, e)
    tpart, dis, inv = sc_call(edge_index.astype(jnp.int32))
    return final_call(tpart, dis, inv, x, W, b.reshape(1, -1))
